# baseline (speedup 1.0000x reference)
import jax
import jax.numpy as jnp
from jax.experimental import pallas as pl
from jax.experimental.pallas import tpu as pltpu

_B = 64
_K = 16384
_N = 4096
_NB = 256
_KCH = 2048
_KPN = _K // _KCH
_TOT = (_N // _NB) * _KPN
_NBUF = 6


def _body(x_ref, b_ref, w_hbm, o_ref, xb_ref, xf_ref, buf_ref, sem_ref, xsem):
    def issue(c, slot):
        n = c // _KPN
        k = jax.lax.rem(c, _KPN)
        pltpu.make_async_copy(
            w_hbm.at[pl.ds(n * _NB, _NB), pl.ds(k * _KCH, _KCH)],
            buf_ref.at[slot],
            sem_ref.at[slot],
        ).start()

    for j in range(_NBUF):
        issue(j, j)
    xcp = pltpu.make_async_copy(x_ref, xf_ref, xsem)
    xcp.start()
    xcp.wait()
    xb_ref[...] = xf_ref[...].astype(jnp.bfloat16)

    def step(c, carry):
        slot = jax.lax.rem(c, _NBUF)
        n = c // _KPN
        k = jax.lax.rem(c, _KPN)
        pltpu.make_async_copy(
            w_hbm.at[pl.ds(n * _NB, _NB), pl.ds(k * _KCH, _KCH)],
            buf_ref.at[slot],
            sem_ref.at[slot],
        ).wait()
        wb = buf_ref[slot].astype(jnp.bfloat16)
        xb = xb_ref[:, pl.ds(k * _KCH, _KCH)]
        part = jax.lax.dot_general(
            xb, wb, (((1,), (1,)), ((), ())),
            preferred_element_type=jnp.float32)
        col = pl.ds(n * _NB, _NB)

        @pl.when(k == 0)
        def _():
            o_ref[:, col] = part + b_ref[:, col]

        @pl.when(k != 0)
        def _():
            o_ref[:, col] = o_ref[:, col] + part

        @pl.when(c + _NBUF < _TOT)
        def _():
            issue(c + _NBUF, slot)

        return carry

    jax.lax.fori_loop(0, _TOT, step, 0)


def kernel(input, weight, bias):
    bias2 = bias.reshape(1, _N)
    return pl.pallas_call(
        _body,
        in_specs=[
            pl.BlockSpec(memory_space=pltpu.MemorySpace.HBM),
            pl.BlockSpec(memory_space=pltpu.MemorySpace.VMEM),
            pl.BlockSpec(memory_space=pltpu.MemorySpace.HBM),
        ],
        out_specs=pl.BlockSpec(memory_space=pltpu.MemorySpace.VMEM),
        out_shape=jax.ShapeDtypeStruct((_B, _N), jnp.float32),
        scratch_shapes=[
            pltpu.VMEM((_B, _K), jnp.bfloat16),
            pltpu.VMEM((_B, _K), jnp.float32),
            pltpu.VMEM((_NBUF, _NB, _KCH), jnp.float32),
            pltpu.SemaphoreType.DMA((_NBUF,)),
            pltpu.SemaphoreType.DMA,
        ],
    )(input, bias2, weight)


# speedup vs baseline: 1.0167x; 1.0167x over previous
import jax
import jax.numpy as jnp
from jax.experimental import pallas as pl
from jax.experimental.pallas import tpu as pltpu

_B = 64
_K = 16384
_N = 4096
_NB = 256
_KCH = 2048
_KPN = _K // _KCH
_TOT = (_N // _NB) * _KPN
_NBUF = 6


def _body(x_ref, b_ref, w_hbm, o_ref, xb_ref, buf_ref, sem_ref):
    def issue(c, slot):
        n = c // _KPN
        k = jax.lax.rem(c, _KPN)
        pltpu.make_async_copy(
            w_hbm.at[pl.ds(n * _NB, _NB), pl.ds(k * _KCH, _KCH)],
            buf_ref.at[slot],
            sem_ref.at[slot],
        ).start()

    for j in range(_NBUF):
        issue(j, j)

    xb_ref[...] = x_ref[...].astype(jnp.bfloat16)

    def step(c, carry):
        slot = jax.lax.rem(c, _NBUF)
        n = c // _KPN
        k = jax.lax.rem(c, _KPN)
        pltpu.make_async_copy(
            w_hbm.at[pl.ds(n * _NB, _NB), pl.ds(k * _KCH, _KCH)],
            buf_ref.at[slot],
            sem_ref.at[slot],
        ).wait()
        wb = buf_ref[slot].astype(jnp.bfloat16)
        xb = xb_ref[:, pl.ds(k * _KCH, _KCH)]
        part = jax.lax.dot_general(
            xb, wb, (((1,), (1,)), ((), ())),
            preferred_element_type=jnp.float32)
        col = pl.ds(n * _NB, _NB)

        @pl.when(k == 0)
        def _():
            o_ref[:, col] = part + b_ref[:, col]

        @pl.when(k != 0)
        def _():
            o_ref[:, col] = o_ref[:, col] + part

        @pl.when(c + _NBUF < _TOT)
        def _():
            issue(c + _NBUF, slot)

        return carry

    jax.lax.fori_loop(0, _TOT, step, 0)


def kernel(input, weight, bias):
    bias2 = bias.reshape(1, _N)
    return pl.pallas_call(
        _body,
        in_specs=[
            pl.BlockSpec(memory_space=pltpu.MemorySpace.VMEM),
            pl.BlockSpec(memory_space=pltpu.MemorySpace.VMEM),
            pl.BlockSpec(memory_space=pltpu.MemorySpace.HBM),
        ],
        out_specs=pl.BlockSpec(memory_space=pltpu.MemorySpace.VMEM),
        out_shape=jax.ShapeDtypeStruct((_B, _N), jnp.float32),
        scratch_shapes=[
            pltpu.VMEM((_B, _K), jnp.bfloat16),
            pltpu.VMEM((_NBUF, _NB, _KCH), jnp.float32),
            pltpu.SemaphoreType.DMA((_NBUF,)),
        ],
    )(input, bias2, weight)


# speedup vs baseline: 1.0191x; 1.0024x over previous
import jax
import jax.numpy as jnp
from jax.experimental import pallas as pl
from jax.experimental.pallas import tpu as pltpu

_B = 64
_K = 16384
_N = 4096
_NB = 256
_KCH = 2048
_KPN = _K // _KCH
_TOT = (_N // _NB) * _KPN
_NBUF = 6


def _body(x_ref, b_ref, w_hbm, o_ref, xb_ref, buf_ref, sem_ref):
    def issue(c, slot):
        n = c // _KPN
        k = jax.lax.rem(c, _KPN)
        pltpu.make_async_copy(
            w_hbm.at[pl.ds(n * _NB, _NB), pl.ds(k * _KCH, _KCH)],
            buf_ref.at[slot],
            sem_ref.at[slot],
        ).start()

    for j in range(_NBUF):
        issue(j, j)

    xb_ref[...] = x_ref[...].astype(jnp.bfloat16)

    _H = _KCH // 2

    def step(c, acc):
        slot = jax.lax.rem(c, _NBUF)
        n = c // _KPN
        k = jax.lax.rem(c, _KPN)
        pltpu.make_async_copy(
            w_hbm.at[pl.ds(n * _NB, _NB), pl.ds(k * _KCH, _KCH)],
            buf_ref.at[slot],
            sem_ref.at[slot],
        ).wait()
        wb0 = buf_ref[slot][:, :_H].astype(jnp.bfloat16)
        wb1 = buf_ref[slot][:, _H:].astype(jnp.bfloat16)
        xb0 = xb_ref[:, pl.ds(k * _KCH, _H)]
        xb1 = xb_ref[:, pl.ds(k * _KCH + _H, _H)]
        part = jax.lax.dot_general(
            xb0, wb0, (((1,), (1,)), ((), ())),
            preferred_element_type=jnp.float32)
        part = part + jax.lax.dot_general(
            xb1, wb1, (((1,), (1,)), ((), ())),
            preferred_element_type=jnp.float32)

        @pl.when(c + _NBUF < _TOT)
        def _():
            issue(c + _NBUF, slot)

        acc = jnp.where(k == 0, part, acc + part)

        @pl.when(k == _KPN - 1)
        def _():
            col = pl.ds(n * _NB, _NB)
            o_ref[:, col] = acc + b_ref[:, col]

        return acc

    jax.lax.fori_loop(0, _TOT, step,
                      jnp.zeros((_B, _NB), jnp.float32))


def kernel(input, weight, bias):
    bias2 = bias.reshape(1, _N)
    return pl.pallas_call(
        _body,
        in_specs=[
            pl.BlockSpec(memory_space=pltpu.MemorySpace.VMEM),
            pl.BlockSpec(memory_space=pltpu.MemorySpace.VMEM),
            pl.BlockSpec(memory_space=pltpu.MemorySpace.HBM),
        ],
        out_specs=pl.BlockSpec(memory_space=pltpu.MemorySpace.VMEM),
        out_shape=jax.ShapeDtypeStruct((_B, _N), jnp.float32),
        scratch_shapes=[
            pltpu.VMEM((_B, _K), jnp.bfloat16),
            pltpu.VMEM((_NBUF, _NB, _KCH), jnp.float32),
            pltpu.SemaphoreType.DMA((_NBUF,)),
        ],
    )(input, bias2, weight)
